# baseline (speedup 1.0000x reference)
import functools

import jax
import jax.numpy as jnp
from jax import lax
from jax.experimental import pallas as pl
from jax.experimental.pallas import tpu as pltpu
from jax.experimental.pallas import tpu_sc as plsc

NX = 128
NY = 128
PATCH_HW = 3
P = 2 * PATCH_HW
SIGMA = 0.92
I0 = 1000.0
INV_ALPHA = 1.0 / (2.0 ** 0.5 * SIGMA)

BATCH = 1024
NSPOTS = 128
NWORKERS = 32
FPW = BATCH // NWORKERS
NGROUPS = NSPOTS // 16


_ERF_COEF = (1.1283753, -0.37602797, 0.11242295, -0.026191296, 0.0046716467,
             -0.00059765606, 4.7740126e-05, -1.7462387e-06)


def _verf_small(s):
    t = s * s
    acc = _ERF_COEF[-1]
    for c in _ERF_COEF[-2::-1]:
        acc = acc * t + c
    return acc * s


def _round_rtne(x):
    r = (x + 0.5).astype(jnp.int32)
    rf = r.astype(jnp.float32)
    tie = (rf - x) == 0.5
    odd = (r & 1) == 1
    return jnp.where(tie & odd, r - 1, r)


def _load_frame(zloc, f):
    xs = [zloc[f, pl.ds(g * 16, 16)] for g in range(NGROUPS)]
    ys = [zloc[f, pl.ds(NSPOTS + g * 16, 16)] for g in range(NGROUPS)]
    return xs, ys


def _zero_scatter_frame(zloc, canvas, f):
    zero16 = jnp.zeros((16,), jnp.float32)
    zero16i = jnp.zeros((16,), jnp.int32)
    xs, ys = _load_frame(zloc, f)
    for g in range(NGROUPS):
        px = _round_rtne(xs[g]) - PATCH_HW
        py = _round_rtne(ys[g]) - PATCH_HW
        rows = [px + i for i in range(P)]
        cols = [py + j for j in range(P)]
        for i in range(P):
            for j in range(P):
                plsc.store_scatter(canvas, [zero16i, rows[i], cols[j]], zero16)


def _emit_frame(zloc, canvas, f):
    zero16i = jnp.zeros((16,), jnp.int32)
    xs, ys = _load_frame(zloc, f)
    for g in range(NGROUPS):
        x0, y0 = xs[g], ys[g]
        px = _round_rtne(x0) - PATCH_HW
        py = _round_rtne(y0) - PATCH_HW
        ux = (x0 - px.astype(jnp.float32)) * INV_ALPHA
        uy = (y0 - py.astype(jnp.float32)) * INV_ALPHA
        bx = [-1.0] + [_verf_small(((float(i) - 0.5) * INV_ALPHA) - ux)
                       for i in range(1, P + 1)]
        by = [-1.0] + [_verf_small(((float(j) - 0.5) * INV_ALPHA) - uy)
                       for j in range(1, P + 1)]
        lxs = [(bx[i + 1] - bx[i]) * (0.25 * I0) for i in range(P)]
        lys = [by[j + 1] - by[j] for j in range(P)]
        rows = [px + i for i in range(P)]
        cols = [py + j for j in range(P)]
        for i in range(P):
            for j in range(P):
                plsc.addupdate_scatter(
                    canvas, [zero16i, rows[i], cols[j]], lxs[i] * lys[j])


def _decoder_body(z_hbm, out_hbm, zloc, canvas0, canvas1, sem0, sem1):
    nc = 2
    wid = lax.axis_index("s") * nc + lax.axis_index("c")
    pltpu.sync_copy(z_hbm.at[pl.ds(wid * FPW, FPW)], zloc)
    bufs = ((canvas0, sem0), (canvas1, sem1))

    zero16 = jnp.zeros((16,), jnp.float32)

    def zbody(k, _):
        for t in range(8):
            canvas0[0, k, pl.ds(t * 16, 16)] = zero16
            canvas1[0, k, pl.ds(t * 16, 16)] = zero16
        return 0
    lax.fori_loop(0, NX, zbody, 0, unroll=False)

    def pair(t, _):
        for k, (canvas, sem) in enumerate(bufs):
            f = 2 * t + k

            @pl.when(t > 0)
            def _wait_and_clear():
                pltpu.make_async_copy(canvas, out_hbm.at[0], sem).wait()
                _zero_scatter_frame(zloc, canvas, f - 2)

            _emit_frame(zloc, canvas, f)
            pltpu.async_copy(canvas, out_hbm.at[wid * FPW + f], sem)
        return 0

    lax.fori_loop(0, FPW // 2, pair, 0, unroll=False)
    for canvas, sem in bufs:
        pltpu.make_async_copy(canvas, out_hbm.at[0], sem).wait()


def kernel(z):
    mesh = plsc.VectorSubcoreMesh(core_axis_name="c", subcore_axis_name="s")
    run = pl.kernel(
        _decoder_body,
        out_type=jax.ShapeDtypeStruct((BATCH, 1, NX, NY), jnp.float32),
        mesh=mesh,
        compiler_params=pltpu.CompilerParams(needs_layout_passes=False),
        scratch_types=[
            pltpu.VMEM((FPW, 2 * NSPOTS), jnp.float32),
            pltpu.VMEM((1, NX, NY), jnp.float32),
            pltpu.VMEM((1, NX, NY), jnp.float32),
            pltpu.SemaphoreType.DMA,
            pltpu.SemaphoreType.DMA,
        ],
    )
    return run(z)


# speedup vs baseline: 1.1382x; 1.1382x over previous
import functools

import jax
import jax.numpy as jnp
from jax import lax
from jax.experimental import pallas as pl
from jax.experimental.pallas import tpu as pltpu
from jax.experimental.pallas import tpu_sc as plsc

NX = 128
NY = 128
PATCH_HW = 3
P = 2 * PATCH_HW
SIGMA = 0.92
I0 = 1000.0
INV_ALPHA = 1.0 / (2.0 ** 0.5 * SIGMA)

BATCH = 1024
NSPOTS = 128
NWORKERS = 32
FPW = BATCH // NWORKERS
NGROUPS = NSPOTS // 16


_ERF_COEF = (1.1283519, -0.375583, 0.11105316, -0.024620397, 0.0038182875,
             -0.00036072027, 1.5237456e-05)


def _verf_small(s):
    t = s * s
    acc = _ERF_COEF[-1]
    for c in _ERF_COEF[-2::-1]:
        acc = acc * t + c
    return acc * s


def _round_rtne(x):
    r = (x + 0.5).astype(jnp.int32)
    rf = r.astype(jnp.float32)
    tie = (rf - x) == 0.5
    odd = (r & 1) == 1
    return jnp.where(tie & odd, r - 1, r)


def _spot_geometry(zloc, f, g):
    x0 = zloc[f, pl.ds(g * 16, 16)]
    y0 = zloc[f, pl.ds(NSPOTS + g * 16, 16)]
    px = _round_rtne(x0) - PATCH_HW
    py = _round_rtne(y0) - PATCH_HW
    return x0, y0, px, py


def _zero_scatter_frame(zloc, canvas, f):
    zero16 = jnp.zeros((16,), jnp.float32)
    for g in range(NGROUPS):
        _, _, px, py = _spot_geometry(zloc, f, g)
        rows = [px + i for i in range(P)]
        cols = [py + j for j in range(P)]
        for i in range(P):
            for j in range(P):
                plsc.store_scatter(canvas, [rows[i], cols[j]], zero16)


def _emit_frame(zloc, canvas, f):
    for g in range(NGROUPS):
        x0, y0, px, py = _spot_geometry(zloc, f, g)
        ux = (x0 - px.astype(jnp.float32)) * INV_ALPHA
        uy = (y0 - py.astype(jnp.float32)) * INV_ALPHA
        bx = [-1.0] + [_verf_small(((float(i) - 0.5) * INV_ALPHA) - ux)
                       for i in range(1, P + 1)]
        by = [-1.0] + [_verf_small(((float(j) - 0.5) * INV_ALPHA) - uy)
                       for j in range(1, P + 1)]
        lxs = [(bx[i + 1] - bx[i]) * (0.25 * I0) for i in range(P)]
        lys = [by[j + 1] - by[j] for j in range(P)]
        rows = [px + i for i in range(P)]
        cols = [py + j for j in range(P)]
        for i in range(P):
            for j in range(P):
                plsc.addupdate_scatter(
                    canvas, [rows[i], cols[j]], lxs[i] * lys[j])


def _decoder_body(z_hbm, out_hbm, zloc, canvas0, canvas1, sem0, sem1):
    nc = 2
    wid = lax.axis_index("s") * nc + lax.axis_index("c")
    pltpu.sync_copy(z_hbm.at[pl.ds(wid * FPW, FPW)], zloc)
    bufs = ((canvas0, sem0), (canvas1, sem1))

    zero16 = jnp.zeros((16,), jnp.float32)

    def zbody(k, _):
        for t in range(8):
            canvas0[k, pl.ds(t * 16, 16)] = zero16
            canvas1[k, pl.ds(t * 16, 16)] = zero16
        return 0
    lax.fori_loop(0, NX, zbody, 0, unroll=False)

    def pair(t, _):
        for k, (canvas, sem) in enumerate(bufs):
            f = 2 * t + k

            @pl.when(t > 0)
            def _wait_and_clear():
                pltpu.make_async_copy(canvas, out_hbm.at[0, 0], sem).wait()
                _zero_scatter_frame(zloc, canvas, f - 2)

            _emit_frame(zloc, canvas, f)
            pltpu.async_copy(canvas, out_hbm.at[wid * FPW + f, 0], sem)
        return 0

    lax.fori_loop(0, FPW // 2, pair, 0, unroll=False)
    for canvas, sem in bufs:
        pltpu.make_async_copy(canvas, out_hbm.at[0, 0], sem).wait()


def kernel(z):
    mesh = plsc.VectorSubcoreMesh(core_axis_name="c", subcore_axis_name="s")
    run = pl.kernel(
        _decoder_body,
        out_type=jax.ShapeDtypeStruct((BATCH, 1, NX, NY), jnp.float32),
        mesh=mesh,
        compiler_params=pltpu.CompilerParams(needs_layout_passes=False),
        scratch_types=[
            pltpu.VMEM((FPW, 2 * NSPOTS), jnp.float32),
            pltpu.VMEM((NX, NY), jnp.float32),
            pltpu.VMEM((NX, NY), jnp.float32),
            pltpu.SemaphoreType.DMA,
            pltpu.SemaphoreType.DMA,
        ],
    )
    return run(z)
